# baseline (speedup 1.0000x reference)
import functools

import jax
import jax.numpy as jnp
from jax import lax
from jax.experimental import pallas as pl
from jax.experimental.pallas import tpu as pltpu
import jax.experimental.pallas.tpu_sc as plsc

_BINS = 64
_EPS = 1e-8

_NC = 2
_NS = 16
_NW = _NC * _NS

_HIST = 2048
_NSTREAM = 6
_HTOT = _NSTREAM * _HIST

_RPB = 1152
_RPT = _NSTREAM * _RPB


def _lab_u64(raw):
    lin = jnp.where(
        raw > (0.04045 - 0.5) / 0.5,
        jnp.exp(2.4 * jnp.log(raw * (0.5 / 1.055) + (0.555 / 1.055))),
        raw * (0.5 / 12.92) + (0.5 / 12.92),
    )
    r, g, b = lin[0], lin[1], lin[2]
    xn = (0.412453 / 0.95047) * r + (0.357580 / 0.95047) * g + (0.180423 / 0.95047) * b
    yn = 0.212671 * r + 0.715160 * g + 0.072169 * b
    zn = (0.019334 / 1.08883) * r + (0.119193 / 1.08883) * g + (0.950227 / 1.08883) * b

    def f(t):
        return jnp.where(
            t > 0.008856,
            jnp.exp((1.0 / 3.0) * jnp.log(t)),
            7.787 * t + 4.0 / 29.0,
        )

    fx, fy, fz = f(xn), f(yn), f(zn)
    uL = fy * (116.0 * 64.0) - (16.0 * 64.0)
    ua = (fx - fy) * (500.0 * 64.0)
    ub = (fy - fz) * (200.0 * 64.0)
    return uL, ua, ub


def _encode_body(pred_ref, targ_ref, out_ref):
    lane16 = (lax.broadcasted_iota(jnp.int32, (1, 384), 1) % 16).astype(jnp.float32)
    just_below_64 = 63.99999618530273
    for t_i, ref in ((0, pred_ref), (1, targ_ref)):
        labs = _lab_u64(ref[0])
        for ch in range(3):
            u = labs[ch]
            s = t_i * 3 + ch
            inr = (u >= 0.0) & (u <= 64.0)
            idx16 = jnp.floor(jnp.clip(u, 0.0, just_below_64)) * 16.0
            base = float(s * _HIST) + lane16
            addr = (base + jnp.where(inr, idx16, 1024.0)).astype(jnp.int32)
            for kt in range(3):
                out_ref[0, s, kt * 384:(kt + 1) * 384, :] = (
                    addr[:, kt * 128:(kt + 1) * 128])


@functools.partial(jax.jit, static_argnums=(2, 3))
def _encode(p, t, i0, nb):
    H, W = p.shape[2], p.shape[3]
    R = H * W // 128
    return pl.pallas_call(
        _encode_body,
        grid=(nb,),
        in_specs=[
            pl.BlockSpec((1, 3, H, W), lambda i: (i0 + i, 0, 0, 0)),
            pl.BlockSpec((1, 3, H, W), lambda i: (i0 + i, 0, 0, 0)),
        ],
        out_specs=pl.BlockSpec((1, _NSTREAM, R, 128), lambda i: (i, 0, 0, 0)),
        out_shape=jax.ShapeDtypeStruct((nb, _NSTREAM, R, 128), jnp.int32),
        compiler_params=pltpu.CompilerParams(dimension_semantics=("arbitrary",)),
    )(p, t)


@functools.lru_cache(maxsize=None)
def _make_sc_hist(rows_per_tile):
    nq = 4
    chunk = rows_per_tile // nq
    assert nq * chunk == rows_per_tile

    def body(idx_hbm, out_hbm, buf0, buf1, hist_v, sem0, sem1):
        cid = lax.axis_index("c")
        sid = lax.axis_index("s")
        wid = sid * _NC + cid

        def zero_body(i, _):
            hist_v[pl.ds(i * 16, 16)] = jnp.zeros((16,), jnp.float32)
            return 0

        lax.fori_loop(0, _HTOT // 16, zero_body, 0)

        ones = jnp.ones((16,), jnp.float32)
        bufs = (buf0, buf1)
        sems = (sem0, sem1)

        def start(q, buf, sem):
            return pltpu.async_copy(
                idx_hbm.at[wid, pl.ds(q * chunk, chunk)], buf, sem)

        def process(buf):
            def row_body(r, _):
                ivs = [buf[r, pl.ds(g * 16, 16)] for g in range(8)]
                for iv in ivs:
                    plsc.addupdate_scatter(hist_v, [iv], ones)
                return 0

            lax.fori_loop(0, chunk, row_body, 0, unroll=8)

        descs = [None, None]
        descs[0] = start(0, bufs[0], sems[0])
        for q in range(nq):
            cur = q % 2
            if q + 1 < nq:
                descs[1 - cur] = start(q + 1, bufs[1 - cur], sems[1 - cur])
            descs[cur].wait()
            process(bufs[cur])

        pltpu.sync_copy(hist_v, out_hbm.at[wid])

    mesh = plsc.VectorSubcoreMesh(core_axis_name="c", subcore_axis_name="s",
                                  num_cores=_NC, num_subcores=_NS)
    return pl.kernel(
        body,
        out_type=jax.ShapeDtypeStruct((_NW, _HTOT), jnp.float32),
        mesh=mesh,
        scratch_types=[
            pltpu.VMEM((chunk, 128), jnp.int32),
            pltpu.VMEM((chunk, 128), jnp.int32),
            pltpu.VMEM((_HTOT,), jnp.float32),
            pltpu.SemaphoreType.DMA,
            pltpu.SemaphoreType.DMA,
        ],
        compiler_params=pltpu.CompilerParams(needs_layout_passes=False),
    )


def _sc_hist(enc):
    return _make_sc_hist(enc.shape[1])(enc)


def _loss_body(hist_ref, out_ref):
    h = hist_ref[...]
    partial = jnp.sum(h, axis=0)
    rows = lax.broadcasted_iota(jnp.int32, (96, 128), 0)
    cols = lax.broadcasted_iota(jnp.int32, (96, 128), 1)
    binmap = (rows % 16) * 8 + cols // 16
    stream = rows // 16

    cdf = []
    for s in range(_NSTREAM):
        part_s = jnp.where(stream == s, partial, 0.0)
        cdf.append([jnp.sum(jnp.where(binmap <= b, part_s, 0.0))
                    for b in range(_BINS)])

    total = 0.0
    for ch in range(3):
        sp = cdf[ch][_BINS - 1]
        st = cdf[3 + ch][_BINS - 1]
        sp = jnp.where(sp == 0.0, _EPS, sp)
        st = jnp.where(st == 0.0, _EPS, st)
        csum = 0.0
        for b in range(_BINS):
            csum += jnp.abs(cdf[ch][b] / sp - cdf[3 + ch][b] / st)
        total += csum / _BINS
    out_ref[0, 0] = total / 3.0


@jax.jit
def _loss(hist):
    out = pl.pallas_call(
        _loss_body,
        out_specs=pl.BlockSpec(memory_space=pltpu.SMEM),
        out_shape=jax.ShapeDtypeStruct((1, 1), jnp.float32),
    )(hist)
    return out[0, 0]


_NPIPE = 8


def kernel(pred, target):
    pred = pred.astype(jnp.float32)
    target = target.astype(jnp.float32)
    B = pred.shape[0]
    g = B // _NPIPE
    hists = []
    for i in range(_NPIPE):
        enc = _encode(pred, target, i * g, g)
        hists.append(_sc_hist(enc.reshape(_NW, (g * _RPT) // _NW, 128)))
    hist = jnp.concatenate(hists, axis=0)
    return _loss(hist.reshape(_NPIPE * _NW, _HTOT // 128, 128))


# speedup vs baseline: 1.1418x; 1.1418x over previous
import functools

import jax
import jax.numpy as jnp
from jax import lax
from jax.experimental import pallas as pl
from jax.experimental.pallas import tpu as pltpu
import jax.experimental.pallas.tpu_sc as plsc

_BINS = 64
_EPS = 1e-8

_NC = 2
_NS = 16
_NW = _NC * _NS

_HIST = 2048
_NSTREAM = 6
_HTOT = _NSTREAM * _HIST

_RPB = 1152
_RPT = _NSTREAM * _RPB


def _lab_u64(raw):
    lin = jnp.where(
        raw > (0.04045 - 0.5) / 0.5,
        jnp.exp(2.4 * jnp.log(raw * (0.5 / 1.055) + (0.555 / 1.055))),
        raw * (0.5 / 12.92) + (0.5 / 12.92),
    )
    r, g, b = lin[0], lin[1], lin[2]
    xn = (0.412453 / 0.95047) * r + (0.357580 / 0.95047) * g + (0.180423 / 0.95047) * b
    yn = 0.212671 * r + 0.715160 * g + 0.072169 * b
    zn = (0.019334 / 1.08883) * r + (0.119193 / 1.08883) * g + (0.950227 / 1.08883) * b

    def f(t):
        return jnp.where(
            t > 0.008856,
            jnp.exp((1.0 / 3.0) * jnp.log(t)),
            7.787 * t + 4.0 / 29.0,
        )

    fx, fy, fz = f(xn), f(yn), f(zn)
    uL = fy * (116.0 * 64.0) - (16.0 * 64.0)
    ua = (fx - fy) * (500.0 * 64.0)
    ub = (fy - fz) * (200.0 * 64.0)
    return uL, ua, ub


def _encode_body(pred_ref, targ_ref, out_ref):
    lane16 = (lax.broadcasted_iota(jnp.int32, (1, 384), 1) % 16).astype(jnp.float32)
    just_below_64 = 63.99999618530273
    for t_i, ref in ((0, pred_ref), (1, targ_ref)):
        labs = _lab_u64(ref[0])
        for ch in range(3):
            u = labs[ch]
            s = t_i * 3 + ch
            inr = (u >= 0.0) & (u <= 64.0)
            idx16 = jnp.floor(jnp.clip(u, 0.0, just_below_64)) * 16.0
            base = float(s * _HIST) + lane16
            addr = (base + jnp.where(inr, idx16, 1024.0)).astype(jnp.int32)
            for kt in range(3):
                out_ref[0, s, kt * 384:(kt + 1) * 384, :] = (
                    addr[:, kt * 128:(kt + 1) * 128])


@functools.partial(jax.jit, static_argnums=(2, 3))
def _encode(p, t, i0, nb):
    H, W = p.shape[2], p.shape[3]
    R = H * W // 128
    return pl.pallas_call(
        _encode_body,
        grid=(nb,),
        in_specs=[
            pl.BlockSpec((1, 3, H, W), lambda i: (i0 + i, 0, 0, 0)),
            pl.BlockSpec((1, 3, H, W), lambda i: (i0 + i, 0, 0, 0)),
        ],
        out_specs=pl.BlockSpec((1, _NSTREAM, R, 128), lambda i: (i, 0, 0, 0)),
        out_shape=jax.ShapeDtypeStruct((nb, _NSTREAM, R, 128), jnp.int32),
        compiler_params=pltpu.CompilerParams(dimension_semantics=("arbitrary",)),
    )(p, t)


@functools.lru_cache(maxsize=None)
def _make_sc_hist(rows_per_tile):
    nq = 4
    chunk = rows_per_tile // nq
    assert nq * chunk == rows_per_tile

    def body(idx_hbm, out_hbm, buf0, buf1, hist_v, sem0, sem1):
        cid = lax.axis_index("c")
        sid = lax.axis_index("s")
        wid = sid * _NC + cid

        def zero_body(i, _):
            hist_v[pl.ds(i * 16, 16)] = jnp.zeros((16,), jnp.float32)
            return 0

        lax.fori_loop(0, _HTOT // 16, zero_body, 0)

        ones = jnp.ones((16,), jnp.float32)
        bufs = (buf0, buf1)
        sems = (sem0, sem1)

        def start(q, buf, sem):
            return pltpu.async_copy(
                idx_hbm.at[wid, pl.ds(q * chunk, chunk)], buf, sem)

        def process(buf):
            def row_body(r, _):
                ivs = [buf[r, pl.ds(g * 16, 16)] for g in range(8)]
                for iv in ivs:
                    plsc.addupdate_scatter(hist_v, [iv], ones)
                return 0

            lax.fori_loop(0, chunk, row_body, 0, unroll=8)

        descs = [None, None]
        descs[0] = start(0, bufs[0], sems[0])
        for q in range(nq):
            cur = q % 2
            if q + 1 < nq:
                descs[1 - cur] = start(q + 1, bufs[1 - cur], sems[1 - cur])
            descs[cur].wait()
            process(bufs[cur])

        pltpu.sync_copy(hist_v, out_hbm.at[wid])

    mesh = plsc.VectorSubcoreMesh(core_axis_name="c", subcore_axis_name="s",
                                  num_cores=_NC, num_subcores=_NS)
    return pl.kernel(
        body,
        out_type=jax.ShapeDtypeStruct((_NW, _HTOT), jnp.float32),
        mesh=mesh,
        scratch_types=[
            pltpu.VMEM((chunk, 128), jnp.int32),
            pltpu.VMEM((chunk, 128), jnp.int32),
            pltpu.VMEM((_HTOT,), jnp.float32),
            pltpu.SemaphoreType.DMA,
            pltpu.SemaphoreType.DMA,
        ],
        compiler_params=pltpu.CompilerParams(needs_layout_passes=False),
    )


def _sc_hist(enc):
    return _make_sc_hist(enc.shape[1])(enc)


def _loss_body(hist_ref, out_ref):
    h = hist_ref[...]
    partial = jnp.sum(h, axis=0)
    rows = lax.broadcasted_iota(jnp.int32, (96, 128), 0)
    cols = lax.broadcasted_iota(jnp.int32, (96, 128), 1)
    binmap = (rows % 16) * 8 + cols // 16
    stream = rows // 16

    cdf = []
    for s in range(_NSTREAM):
        part_s = jnp.where(stream == s, partial, 0.0)
        cdf.append([jnp.sum(jnp.where(binmap <= b, part_s, 0.0))
                    for b in range(_BINS)])

    total = 0.0
    for ch in range(3):
        sp = cdf[ch][_BINS - 1]
        st = cdf[3 + ch][_BINS - 1]
        sp = jnp.where(sp == 0.0, _EPS, sp)
        st = jnp.where(st == 0.0, _EPS, st)
        csum = 0.0
        for b in range(_BINS):
            csum += jnp.abs(cdf[ch][b] / sp - cdf[3 + ch][b] / st)
        total += csum / _BINS
    out_ref[0, 0] = total / 3.0


@jax.jit
def _loss(hist):
    out = pl.pallas_call(
        _loss_body,
        out_specs=pl.BlockSpec(memory_space=pltpu.SMEM),
        out_shape=jax.ShapeDtypeStruct((1, 1), jnp.float32),
    )(hist)
    return out[0, 0]


_NPIPE = 4


def kernel(pred, target):
    pred = pred.astype(jnp.float32)
    target = target.astype(jnp.float32)
    B = pred.shape[0]
    g = B // _NPIPE
    hists = []
    for i in range(_NPIPE):
        enc = _encode(pred, target, i * g, g)
        hists.append(_sc_hist(enc.reshape(_NW, (g * _RPT) // _NW, 128)))
    hist = jnp.concatenate(hists, axis=0)
    return _loss(hist.reshape(_NPIPE * _NW, _HTOT // 128, 128))


# speedup vs baseline: 1.1477x; 1.0052x over previous
import functools

import jax
import jax.numpy as jnp
from jax import lax
from jax.experimental import pallas as pl
from jax.experimental.pallas import tpu as pltpu
import jax.experimental.pallas.tpu_sc as plsc

_BINS = 64
_EPS = 1e-8

_NC = 2
_NS = 16
_NW = _NC * _NS

_HIST = 2048
_NSTREAM = 6
_HTOT = _NSTREAM * _HIST

_RPB = 1152
_RPT = _NSTREAM * _RPB


def _lab_u64(raw):
    lin = jnp.where(
        raw > (0.04045 - 0.5) / 0.5,
        jnp.exp2(2.4 * jnp.log2(raw * (0.5 / 1.055) + (0.555 / 1.055))),
        raw * (0.5 / 12.92) + (0.5 / 12.92),
    )
    r, g, b = lin[0], lin[1], lin[2]
    xn = (0.412453 / 0.95047) * r + (0.357580 / 0.95047) * g + (0.180423 / 0.95047) * b
    yn = 0.212671 * r + 0.715160 * g + 0.072169 * b
    zn = (0.019334 / 1.08883) * r + (0.119193 / 1.08883) * g + (0.950227 / 1.08883) * b

    def f(t):
        return jnp.where(
            t > 0.008856,
            jnp.exp2((1.0 / 3.0) * jnp.log2(t)),
            7.787 * t + 4.0 / 29.0,
        )

    fx, fy, fz = f(xn), f(yn), f(zn)
    uL = fy * (116.0 * 64.0) - (16.0 * 64.0)
    ua = (fx - fy) * (500.0 * 64.0)
    ub = (fy - fz) * (200.0 * 64.0)
    return uL, ua, ub


def _encode_body(pred_ref, targ_ref, out_ref):
    lane16 = (lax.broadcasted_iota(jnp.int32, (1, 384), 1) % 16).astype(jnp.float32)
    just_below_64 = 63.99999618530273
    for t_i, ref in ((0, pred_ref), (1, targ_ref)):
        labs = _lab_u64(ref[0])
        for ch in range(3):
            u = labs[ch]
            s = t_i * 3 + ch
            inr = (u >= 0.0) & (u <= 64.0)
            idx16 = jnp.floor(jnp.minimum(u, just_below_64)) * 16.0
            base = float(s * _HIST) + lane16
            addr = (base + jnp.where(inr, idx16, 1024.0)).astype(jnp.int32)
            for kt in range(3):
                out_ref[0, s, kt * 384:(kt + 1) * 384, :] = (
                    addr[:, kt * 128:(kt + 1) * 128])


@functools.partial(jax.jit, static_argnums=(2, 3))
def _encode(p, t, i0, nb):
    H, W = p.shape[2], p.shape[3]
    R = H * W // 128
    return pl.pallas_call(
        _encode_body,
        grid=(nb,),
        in_specs=[
            pl.BlockSpec((1, 3, H, W), lambda i: (i0 + i, 0, 0, 0)),
            pl.BlockSpec((1, 3, H, W), lambda i: (i0 + i, 0, 0, 0)),
        ],
        out_specs=pl.BlockSpec((1, _NSTREAM, R, 128), lambda i: (i, 0, 0, 0)),
        out_shape=jax.ShapeDtypeStruct((nb, _NSTREAM, R, 128), jnp.int32),
        compiler_params=pltpu.CompilerParams(dimension_semantics=("arbitrary",)),
    )(p, t)


@functools.lru_cache(maxsize=None)
def _make_sc_hist(rows_per_tile):
    nq = 4
    chunk = rows_per_tile // nq
    assert nq * chunk == rows_per_tile

    def body(idx_hbm, out_hbm, buf0, buf1, hist_v, sem0, sem1):
        cid = lax.axis_index("c")
        sid = lax.axis_index("s")
        wid = sid * _NC + cid

        def zero_body(i, _):
            hist_v[pl.ds(i * 16, 16)] = jnp.zeros((16,), jnp.float32)
            return 0

        lax.fori_loop(0, _HTOT // 16, zero_body, 0)

        ones = jnp.ones((16,), jnp.float32)
        bufs = (buf0, buf1)
        sems = (sem0, sem1)

        def start(q, buf, sem):
            return pltpu.async_copy(
                idx_hbm.at[wid, pl.ds(q * chunk, chunk)], buf, sem)

        def process(buf):
            def row_body(r, _):
                ivs = [buf[r, pl.ds(g * 16, 16)] for g in range(8)]
                for iv in ivs:
                    plsc.addupdate_scatter(hist_v, [iv], ones)
                return 0

            lax.fori_loop(0, chunk, row_body, 0, unroll=8)

        descs = [None, None]
        descs[0] = start(0, bufs[0], sems[0])
        for q in range(nq):
            cur = q % 2
            if q + 1 < nq:
                descs[1 - cur] = start(q + 1, bufs[1 - cur], sems[1 - cur])
            descs[cur].wait()
            process(bufs[cur])

        pltpu.sync_copy(hist_v, out_hbm.at[wid])

    mesh = plsc.VectorSubcoreMesh(core_axis_name="c", subcore_axis_name="s",
                                  num_cores=_NC, num_subcores=_NS)
    return pl.kernel(
        body,
        out_type=jax.ShapeDtypeStruct((_NW, _HTOT), jnp.float32),
        mesh=mesh,
        scratch_types=[
            pltpu.VMEM((chunk, 128), jnp.int32),
            pltpu.VMEM((chunk, 128), jnp.int32),
            pltpu.VMEM((_HTOT,), jnp.float32),
            pltpu.SemaphoreType.DMA,
            pltpu.SemaphoreType.DMA,
        ],
        compiler_params=pltpu.CompilerParams(needs_layout_passes=False),
    )


def _sc_hist(enc):
    return _make_sc_hist(enc.shape[1])(enc)


def _loss_body(hist_ref, out_ref):
    h = hist_ref[...]
    partial = jnp.sum(h, axis=0)
    rows = lax.broadcasted_iota(jnp.int32, (96, 128), 0)
    cols = lax.broadcasted_iota(jnp.int32, (96, 128), 1)
    binmap = (rows % 16) * 8 + cols // 16
    stream = rows // 16

    cdf = []
    for s in range(_NSTREAM):
        part_s = jnp.where(stream == s, partial, 0.0)
        cdf.append([jnp.sum(jnp.where(binmap <= b, part_s, 0.0))
                    for b in range(_BINS)])

    total = 0.0
    for ch in range(3):
        sp = cdf[ch][_BINS - 1]
        st = cdf[3 + ch][_BINS - 1]
        sp = jnp.where(sp == 0.0, _EPS, sp)
        st = jnp.where(st == 0.0, _EPS, st)
        csum = 0.0
        for b in range(_BINS):
            csum += jnp.abs(cdf[ch][b] / sp - cdf[3 + ch][b] / st)
        total += csum / _BINS
    out_ref[0, 0] = total / 3.0


@jax.jit
def _loss(hist):
    out = pl.pallas_call(
        _loss_body,
        out_specs=pl.BlockSpec(memory_space=pltpu.SMEM),
        out_shape=jax.ShapeDtypeStruct((1, 1), jnp.float32),
    )(hist)
    return out[0, 0]


_NPIPE = 4


def kernel(pred, target):
    pred = pred.astype(jnp.float32)
    target = target.astype(jnp.float32)
    B = pred.shape[0]
    g = B // _NPIPE
    hists = []
    for i in range(_NPIPE):
        enc = _encode(pred, target, i * g, g)
        hists.append(_sc_hist(enc.reshape(_NW, (g * _RPT) // _NW, 128)))
    hist = jnp.concatenate(hists, axis=0)
    return _loss(hist.reshape(_NPIPE * _NW, _HTOT // 128, 128))
